# baseline (speedup 1.0000x reference)
import functools

import jax
import jax.numpy as jnp
from jax import lax
from jax.experimental import pallas as pl
from jax.experimental.pallas import tpu as pltpu


def _gat_layer(xb, w4_ref, asrc_ref, adst_ref, bias_ref, adj_ref,
               awd_ref, aww_ref, ab_ref, mask_ref, pb_refs,
               *, n_gat, f_out, residual, act, gat_merge):
    hb = 6 * n_gat
    n = xb.shape[0]
    f32 = jnp.float32

    w4 = w4_ref[0]
    wb4 = w4.astype(jnp.bfloat16)
    wad_cols = [lax.dot_general(w4[b], adst_ref[0, b], (((1,), (1,)), ((), ())),
                                preferred_element_type=f32) for b in range(hb)]
    was_cols = [lax.dot_general(w4[b], asrc_ref[0, b], (((1,), (1,)), ((), ())),
                                preferred_element_type=f32) for b in range(hb)]
    wad = jnp.concatenate(wad_cols, axis=1).astype(jnp.bfloat16)
    was = jnp.concatenate(was_cols, axis=1).astype(jnp.bfloat16)

    whs = [jnp.dot(xb, wb4[b], preferred_element_type=f32) for b in range(hb)]
    sdst = jnp.dot(xb, wad, preferred_element_type=f32)
    ssrc_t = lax.dot_general(was, xb, (((0,), (1,)), ((), ())),
                             preferred_element_type=f32)

    maxs = jnp.max(ssrc_t, axis=1, keepdims=True)
    sm = ssrc_t - maxs
    b1 = jnp.exp(sm).astype(jnp.bfloat16)
    b2 = jnp.exp(0.2 * sm).astype(jnp.bfloat16)
    u = sdst + jnp.broadcast_to(maxs.T, sdst.shape)
    a1 = jnp.exp(0.8 * jnp.minimum(u, 0.0)).astype(jnp.bfloat16)
    a2 = jnp.exp(-0.8 * jnp.maximum(u, 0.0)).astype(jnp.bfloat16)

    ones_blk = jnp.ones((n, 128 - f_out), jnp.bfloat16)
    bias2 = bias_ref[0]

    head_outs = []
    cb_n = n // 128
    for c in range(6):
        for g in range(n_gat):
            idx = c * n_gat + g
            pb = pb_refs[idx % 2]
            for rb in range(cb_n):
                r0 = rb * 128
                a1c = a1[r0:r0 + 128, idx:idx + 1]
                a2c = a2[r0:r0 + 128, idx:idx + 1]
                for cb in range(cb_n):
                    c0 = cb * 128
                    pb[r0:r0 + 128, c0:c0 + 128] = (
                        jnp.maximum(a1c * b1[idx:idx + 1, c0:c0 + 128],
                                    a2c * b2[idx:idx + 1, c0:c0 + 128])
                        * adj_ref[c][r0:r0 + 128, c0:c0 + 128])
            rhs = jnp.concatenate([whs[idx].astype(jnp.bfloat16), ones_blk],
                                  axis=1)
            num_ext = jnp.dot(pb[...], rhs,
                              preferred_element_type=f32)
            denom = num_ext[:, f_out:f_out + 1]
            inv = pl.reciprocal(jnp.maximum(denom, f32(1e-20)), approx=True)
            ho = num_ext[:, :f_out] * inv
            if residual:
                ho = ho + whs[idx]
            head_outs.append(ho + bias2[idx:idx + 1, :])

    slab = jnp.concatenate(head_outs, axis=1)
    if act == "elu":
        slab = jnp.where(slab > 0, slab, jnp.exp(slab) - 1.0)

    if gat_merge == "cat":
        cw = n_gat * f_out
        fc = cw
        chans = [slab[:, c * cw:(c + 1) * cw] for c in range(6)]
    else:
        fc = f_out
        inv_g = f32(1.0 / n_gat)
        chans = []
        for c in range(6):
            acc = slab[:, (c * n_gat) * f_out:(c * n_gat + 1) * f_out]
            for g in range(1, n_gat):
                lo = (c * n_gat + g) * f_out
                acc = acc + slab[:, lo:lo + f_out]
            chans.append(acc * inv_g)

    tawd = awd_ref[0].T
    taww = aww_ref[0].T
    jcol = lax.broadcasted_iota(jnp.int32, (fc, 6), 1) // 2
    blocks = []
    for c in range(6):
        sel_w = tawd if c % 2 == 0 else taww
        blocks.append(jnp.where(jcol == (c // 2), sel_w, f32(0.0)))
    gmat = jnp.concatenate(blocks, axis=0)
    cat_c = slab if gat_merge == "cat" else jnp.concatenate(chans, axis=1)
    logits = (jnp.dot(cat_c, gmat, preferred_element_type=f32)
              + ab_ref[0])
    agg = []
    for k in range(3):
        a_c = chans[2 * k]
        b_c = chans[2 * k + 1]
        l0 = logits[:, 2 * k:2 * k + 1]
        l1 = logits[:, 2 * k + 1:2 * k + 2]
        m2 = jnp.maximum(l0, l1)
        e0 = jnp.exp(l0 - m2)
        e1 = jnp.exp(l1 - m2)
        inv2 = pl.reciprocal(e0 + e1, approx=True)
        agg.append(a_c * (e0 * inv2) + b_c * (e1 * inv2))

    mask = mask_ref[...]
    sel = jnp.where(mask[:, 0:1] > 0, agg[1], agg[0])
    sel = jnp.where(mask[:, 1:2] > 0, agg[2], sel)
    return sel


def _net_kernel(x_ref, adj_ref, mask_ref,
                w1_ref, as1_ref, ad1_ref, b1_ref, awd1_ref, aww1_ref, ab1_ref,
                w2_ref, as2_ref, ad2_ref, b2_ref, awd2_ref, aww2_ref, ab2_ref,
                out_ref, h_ref, adjb_ref, pb0_ref, pb1_ref,
                *, n_gat, f1, f2):
    i = pl.program_id(0)

    @pl.when(i == 0)
    def _cast_adj():
        for c in range(6):
            adjb_ref[c] = adj_ref[c].astype(jnp.bfloat16)

    @pl.when(i < 2)
    def _layer1():
        xb = x_ref[...].astype(jnp.bfloat16)
        sel = _gat_layer(xb, w1_ref, as1_ref, ad1_ref, b1_ref, adjb_ref,
                         awd1_ref, aww1_ref, ab1_ref, mask_ref,
                         (pb0_ref, pb1_ref),
                         n_gat=n_gat, f_out=f1, residual=True, act="elu",
                         gat_merge="cat")
        selb = sel.astype(jnp.bfloat16)
        fc1 = n_gat * f1

        @pl.when(i == 0)
        def _():
            h_ref[:, 0:fc1] = selb

        @pl.when(i == 1)
        def _():
            h_ref[:, fc1:2 * fc1] = selb

    @pl.when(i >= 2)
    def _layer2():
        sel = _gat_layer(h_ref[...], w2_ref, as2_ref, ad2_ref, b2_ref,
                         adjb_ref, awd2_ref, aww2_ref, ab2_ref, mask_ref,
                         (pb0_ref, pb1_ref),
                         n_gat=n_gat, f_out=f2, residual=False, act="linear",
                         gat_merge="mean")

        @pl.when(i == 2)
        def _():
            out_ref[...] = sel * 0.5

        @pl.when(i == 3)
        def _():
            out_ref[...] = out_ref[...] + sel * 0.5


def kernel(x, adj, mask2,
           p1_W, p1_a_src, p1_a_dst, p1_bias, p1_aggr_wD, p1_aggr_wW, p1_aggr_b,
           p2_W, p2_a_src, p2_a_dst, p2_bias, p2_aggr_wD, p2_aggr_wW, p2_aggr_b):
    n, f_in1 = x.shape
    nh = 2
    n_gat = 2
    hb = 6 * n_gat
    f1 = p1_W.shape[-1]
    f2 = p2_W.shape[-1]
    f_in2 = p2_W.shape[1]
    fc1 = n_gat * f1

    w1 = p1_W.reshape(nh, hb, f_in1, f1)
    as1 = p1_a_src.reshape(nh, hb, 1, f1)
    ad1 = p1_a_dst.reshape(nh, hb, 1, f1)
    b1 = p1_bias.reshape(nh, hb, f1)
    awd1 = p1_aggr_wD.reshape(nh, 6, fc1)
    aww1 = p1_aggr_wW.reshape(nh, 6, fc1)
    ab1 = p1_aggr_b.reshape(nh, 1, 6)
    w2 = p2_W.reshape(nh, hb, f_in2, f2)
    as2 = p2_a_src.reshape(nh, hb, 1, f2)
    ad2 = p2_a_dst.reshape(nh, hb, 1, f2)
    b2 = p2_bias.reshape(nh, hb, f2)
    awd2 = p2_aggr_wD.reshape(nh, 6, f2)
    aww2 = p2_aggr_wW.reshape(nh, 6, f2)
    ab2 = p2_aggr_b.reshape(nh, 1, 6)

    hsel = lambda i: (i % 2, 0, 0)
    hsel4 = lambda i: (i % 2, 0, 0, 0)
    body = functools.partial(_net_kernel, n_gat=n_gat, f1=f1, f2=f2)
    return pl.pallas_call(
        body,
        out_shape=jax.ShapeDtypeStruct((n, f2), jnp.float32),
        grid=(2 * nh,),
        in_specs=[
            pl.BlockSpec((n, f_in1), lambda i: (0, 0)),
            pl.BlockSpec((6, n, n), lambda i: (0, 0, 0)),
            pl.BlockSpec((n, 2), lambda i: (0, 0)),
            pl.BlockSpec((1, hb, f_in1, f1), hsel4),
            pl.BlockSpec((1, hb, 1, f1), hsel4),
            pl.BlockSpec((1, hb, 1, f1), hsel4),
            pl.BlockSpec((1, hb, f1), hsel),
            pl.BlockSpec((1, 6, fc1), hsel),
            pl.BlockSpec((1, 6, fc1), hsel),
            pl.BlockSpec((1, 1, 6), hsel),
            pl.BlockSpec((1, hb, f_in2, f2), hsel4),
            pl.BlockSpec((1, hb, 1, f2), hsel4),
            pl.BlockSpec((1, hb, 1, f2), hsel4),
            pl.BlockSpec((1, hb, f2), hsel),
            pl.BlockSpec((1, 6, f2), hsel),
            pl.BlockSpec((1, 6, f2), hsel),
            pl.BlockSpec((1, 1, 6), hsel),
        ],
        out_specs=pl.BlockSpec((n, f2), lambda i: (0, 0)),
        scratch_shapes=[pltpu.VMEM((n, f_in2), jnp.bfloat16),
                        pltpu.VMEM((6, n, n), jnp.bfloat16),
                        pltpu.VMEM((n, n), jnp.bfloat16),
                        pltpu.VMEM((n, n), jnp.bfloat16)],
        compiler_params=pltpu.CompilerParams(
            dimension_semantics=("arbitrary",)),
    )(x, adj, mask2, w1, as1, ad1, b1, awd1, aww1, ab1,
      w2, as2, ad2, b2, awd2, aww2, ab2)


# speedup vs baseline: 1.2124x; 1.2124x over previous
import functools

import jax
import jax.numpy as jnp
from jax import lax
from jax.experimental import pallas as pl
from jax.experimental.pallas import tpu as pltpu


def _gat_layer(xb, w4_ref, asrc_ref, adst_ref, bias_ref, adj_ref,
               awd_ref, aww_ref, ab_ref, mask_ref,
               *, n_gat, f_out, residual, act, gat_merge):
    hb = 6 * n_gat
    n = xb.shape[0]
    f32 = jnp.float32

    w4 = w4_ref[0]
    wb4 = w4.astype(jnp.bfloat16)
    wad_cols = [lax.dot_general(w4[b], adst_ref[0, b], (((1,), (1,)), ((), ())),
                                preferred_element_type=f32) for b in range(hb)]
    was_cols = [lax.dot_general(w4[b], asrc_ref[0, b], (((1,), (1,)), ((), ())),
                                preferred_element_type=f32) for b in range(hb)]
    wad = jnp.concatenate(wad_cols, axis=1).astype(jnp.bfloat16)
    was = jnp.concatenate(was_cols, axis=1).astype(jnp.bfloat16)

    whs = [jnp.dot(xb, wb4[b], preferred_element_type=f32) for b in range(hb)]
    sdst = jnp.dot(xb, wad, preferred_element_type=f32)
    ssrc_t = lax.dot_general(was, xb, (((0,), (1,)), ((), ())),
                             preferred_element_type=f32)

    maxs = jnp.max(ssrc_t, axis=1, keepdims=True)
    sm = ssrc_t - maxs
    b1 = jnp.exp(sm).astype(jnp.bfloat16)
    b2 = jnp.exp(0.2 * sm).astype(jnp.bfloat16)
    u = sdst + jnp.broadcast_to(maxs.T, sdst.shape)
    a1 = jnp.exp(0.8 * jnp.minimum(u, 0.0)).astype(jnp.bfloat16)
    a2 = jnp.exp(-0.8 * jnp.maximum(u, 0.0)).astype(jnp.bfloat16)

    ones_blk = jnp.ones((n, 128 - f_out), jnp.bfloat16)
    bias2 = bias_ref[0]

    head_outs = []
    for c in range(6):
        adj_c = adj_ref[c]
        for g in range(n_gat):
            idx = c * n_gat + g
            p1 = a1[:, idx:idx + 1] * b1[idx:idx + 1, :]
            p2 = a2[:, idx:idx + 1] * b2[idx:idx + 1, :]
            p = jnp.maximum(p1, p2) * adj_c
            rhs = jnp.concatenate([whs[idx].astype(jnp.bfloat16), ones_blk],
                                  axis=1)
            num_ext = jnp.dot(p, rhs,
                              preferred_element_type=f32)
            denom = num_ext[:, f_out:f_out + 1]
            inv = pl.reciprocal(jnp.maximum(denom, f32(1e-20)), approx=True)
            ho = num_ext[:, :f_out] * inv
            if residual:
                ho = ho + whs[idx]
            head_outs.append(ho + bias2[idx:idx + 1, :])

    slab = jnp.concatenate(head_outs, axis=1)
    if act == "elu":
        slab = jnp.where(slab > 0, slab, jnp.exp(slab) - 1.0)

    if gat_merge == "cat":
        cw = n_gat * f_out
        fc = cw
        chans = [slab[:, c * cw:(c + 1) * cw] for c in range(6)]
    else:
        fc = f_out
        inv_g = f32(1.0 / n_gat)
        chans = []
        for c in range(6):
            acc = slab[:, (c * n_gat) * f_out:(c * n_gat + 1) * f_out]
            for g in range(1, n_gat):
                lo = (c * n_gat + g) * f_out
                acc = acc + slab[:, lo:lo + f_out]
            chans.append(acc * inv_g)

    tawd = awd_ref[0].T
    taww = aww_ref[0].T
    jcol = lax.broadcasted_iota(jnp.int32, (fc, 6), 1) // 2
    blocks = []
    for c in range(6):
        sel_w = tawd if c % 2 == 0 else taww
        blocks.append(jnp.where(jcol == (c // 2), sel_w, f32(0.0)))
    gmat = jnp.concatenate(blocks, axis=0)
    cat_c = slab if gat_merge == "cat" else jnp.concatenate(chans, axis=1)
    logits = (jnp.dot(cat_c, gmat, preferred_element_type=f32)
              + ab_ref[0])
    agg = []
    for k in range(3):
        a_c = chans[2 * k]
        b_c = chans[2 * k + 1]
        l0 = logits[:, 2 * k:2 * k + 1]
        l1 = logits[:, 2 * k + 1:2 * k + 2]
        m2 = jnp.maximum(l0, l1)
        e0 = jnp.exp(l0 - m2)
        e1 = jnp.exp(l1 - m2)
        inv2 = pl.reciprocal(e0 + e1, approx=True)
        agg.append(a_c * (e0 * inv2) + b_c * (e1 * inv2))

    mask = mask_ref[...]
    sel = jnp.where(mask[:, 0:1] > 0, agg[1], agg[0])
    sel = jnp.where(mask[:, 1:2] > 0, agg[2], sel)
    return sel


def _net_kernel(x_ref, adj_ref, mask_ref,
                w1_ref, as1_ref, ad1_ref, b1_ref, awd1_ref, aww1_ref, ab1_ref,
                w2_ref, as2_ref, ad2_ref, b2_ref, awd2_ref, aww2_ref, ab2_ref,
                out_ref, h_ref, adjb_ref, *, n_gat, f1, f2):
    i = pl.program_id(0)

    @pl.when(i == 0)
    def _cast_adj():
        for c in range(6):
            adjb_ref[c] = adj_ref[c].astype(jnp.bfloat16)

    @pl.when(i < 2)
    def _layer1():
        xb = x_ref[...].astype(jnp.bfloat16)
        sel = _gat_layer(xb, w1_ref, as1_ref, ad1_ref, b1_ref, adjb_ref,
                         awd1_ref, aww1_ref, ab1_ref, mask_ref,
                         n_gat=n_gat, f_out=f1, residual=True, act="elu",
                         gat_merge="cat")
        selb = sel.astype(jnp.bfloat16)
        fc1 = n_gat * f1

        @pl.when(i == 0)
        def _():
            h_ref[:, 0:fc1] = selb

        @pl.when(i == 1)
        def _():
            h_ref[:, fc1:2 * fc1] = selb

    @pl.when(i >= 2)
    def _layer2():
        sel = _gat_layer(h_ref[...], w2_ref, as2_ref, ad2_ref, b2_ref, adjb_ref,
                         awd2_ref, aww2_ref, ab2_ref, mask_ref,
                         n_gat=n_gat, f_out=f2, residual=False, act="linear",
                         gat_merge="mean")

        @pl.when(i == 2)
        def _():
            out_ref[...] = sel * 0.5

        @pl.when(i == 3)
        def _():
            out_ref[...] = out_ref[...] + sel * 0.5


def kernel(x, adj, mask2,
           p1_W, p1_a_src, p1_a_dst, p1_bias, p1_aggr_wD, p1_aggr_wW, p1_aggr_b,
           p2_W, p2_a_src, p2_a_dst, p2_bias, p2_aggr_wD, p2_aggr_wW, p2_aggr_b):
    n, f_in1 = x.shape
    nh = 2
    n_gat = 2
    hb = 6 * n_gat
    f1 = p1_W.shape[-1]
    f2 = p2_W.shape[-1]
    f_in2 = p2_W.shape[1]
    fc1 = n_gat * f1

    w1 = p1_W.reshape(nh, hb, f_in1, f1)
    as1 = p1_a_src.reshape(nh, hb, 1, f1)
    ad1 = p1_a_dst.reshape(nh, hb, 1, f1)
    b1 = p1_bias.reshape(nh, hb, f1)
    awd1 = p1_aggr_wD.reshape(nh, 6, fc1)
    aww1 = p1_aggr_wW.reshape(nh, 6, fc1)
    ab1 = p1_aggr_b.reshape(nh, 1, 6)
    w2 = p2_W.reshape(nh, hb, f_in2, f2)
    as2 = p2_a_src.reshape(nh, hb, 1, f2)
    ad2 = p2_a_dst.reshape(nh, hb, 1, f2)
    b2 = p2_bias.reshape(nh, hb, f2)
    awd2 = p2_aggr_wD.reshape(nh, 6, f2)
    aww2 = p2_aggr_wW.reshape(nh, 6, f2)
    ab2 = p2_aggr_b.reshape(nh, 1, 6)

    hsel = lambda i: (i % 2, 0, 0)
    hsel4 = lambda i: (i % 2, 0, 0, 0)
    body = functools.partial(_net_kernel, n_gat=n_gat, f1=f1, f2=f2)
    return pl.pallas_call(
        body,
        out_shape=jax.ShapeDtypeStruct((n, f2), jnp.float32),
        grid=(2 * nh,),
        in_specs=[
            pl.BlockSpec((n, f_in1), lambda i: (0, 0)),
            pl.BlockSpec((6, n, n), lambda i: (0, 0, 0)),
            pl.BlockSpec((n, 2), lambda i: (0, 0)),
            pl.BlockSpec((1, hb, f_in1, f1), hsel4),
            pl.BlockSpec((1, hb, 1, f1), hsel4),
            pl.BlockSpec((1, hb, 1, f1), hsel4),
            pl.BlockSpec((1, hb, f1), hsel),
            pl.BlockSpec((1, 6, fc1), hsel),
            pl.BlockSpec((1, 6, fc1), hsel),
            pl.BlockSpec((1, 1, 6), hsel),
            pl.BlockSpec((1, hb, f_in2, f2), hsel4),
            pl.BlockSpec((1, hb, 1, f2), hsel4),
            pl.BlockSpec((1, hb, 1, f2), hsel4),
            pl.BlockSpec((1, hb, f2), hsel),
            pl.BlockSpec((1, 6, f2), hsel),
            pl.BlockSpec((1, 6, f2), hsel),
            pl.BlockSpec((1, 1, 6), hsel),
        ],
        out_specs=pl.BlockSpec((n, f2), lambda i: (0, 0)),
        scratch_shapes=[pltpu.VMEM((n, f_in2), jnp.bfloat16),
                        pltpu.VMEM((6, n, n), jnp.bfloat16)],
        compiler_params=pltpu.CompilerParams(
            dimension_semantics=("arbitrary",)),
    )(x, adj, mask2, w1, as1, ad1, b1, awd1, aww1, ab1,
      w2, as2, ad2, b2, awd2, aww2, ab2)
